# baseline (speedup 1.0000x reference)
import functools

import jax
import jax.numpy as jnp
from jax import lax
from jax.experimental import pallas as pl
from jax.experimental.pallas import tpu as pltpu
from jax.experimental.pallas import tpu_sc as plsc

B, R, C = 2, 8192, 2048
NC, NS, L = 2, 16, 16
NW = NC * NS
SLAB = 8
UNROLL = 8

TC_ROWS = 9728
SC_ROWS = B * R - TC_ROWS
ROWS_PER_W = SC_ROWS // NW
N_CHUNKS = ROWS_PER_W // SLAB
TC_BLK = 512


def _sc_partial_kernel(in_hbm, tg_hbm, sums_hbm, cnts_hbm,
                       ib0, tb0, ib1, tb1, ovec, cvec,
                       si0, st0, si1, st1):
    wid = lax.axis_index("s") * NC + lax.axis_index("c")
    fr0 = TC_ROWS + wid * ROWS_PER_W

    zero = jnp.zeros((L,), jnp.float32)
    one = jnp.full((L,), 1.0, jnp.float32)
    acc0 = tuple(zero for _ in range(UNROLL))
    cnt0 = tuple(zero for _ in range(UNROLL))

    def start(c, ib, tb, semi, semt):
        fr = fr0 + c * SLAB
        b = fr // R
        r = fr % R
        pltpu.async_copy(in_hbm.at[b, pl.ds(r, SLAB), :], ib, semi)
        pltpu.async_copy(tg_hbm.at[b, pl.ds(r, SLAB), :], tb, semt)

    def wait(ib, tb, semi, semt):
        pltpu.make_async_copy(in_hbm.at[0, pl.ds(0, SLAB), :], ib, semi).wait()
        pltpu.make_async_copy(tg_hbm.at[0, pl.ds(0, SLAB), :], tb, semt).wait()

    def compute(ib, tb, acc, cnt):
        @plsc.parallel_loop(0, C, step=L, unroll=1, carry=(acc, cnt))
        def inner(j, carry):
            acc2, cnt2 = carry
            acc2 = list(acc2)
            cnt2 = list(cnt2)
            for u in range(UNROLL):
                t = tb[u, pl.ds(j, L)]
                x = ib[u, pl.ds(j, L)]
                d = t - x
                m = t >= 1.0
                acc2[u] = acc2[u] + jnp.where(m, d * d, zero)
                cnt2[u] = cnt2[u] + jnp.where(m, one, zero)
            return tuple(acc2), tuple(cnt2)

        return inner

    start(0, ib0, tb0, si0, st0)
    start(1, ib1, tb1, si1, st1)

    def pair(p, carry):
        acc, cnt = carry
        c0 = 2 * p
        wait(ib0, tb0, si0, st0)
        acc, cnt = compute(ib0, tb0, acc, cnt)

        @pl.when(c0 + 2 < N_CHUNKS)
        def _():
            start(c0 + 2, ib0, tb0, si0, st0)

        wait(ib1, tb1, si1, st1)
        acc, cnt = compute(ib1, tb1, acc, cnt)

        @pl.when(c0 + 3 < N_CHUNKS)
        def _():
            start(c0 + 3, ib1, tb1, si1, st1)

        return acc, cnt

    acc, cnt = lax.fori_loop(0, N_CHUNKS // 2, pair, (acc0, cnt0))

    s = acc[0]
    for u in range(1, UNROLL):
        s = s + acc[u]
    ovec[...] = s
    pltpu.sync_copy(ovec, sums_hbm.at[wid])

    k = cnt[0]
    for u in range(1, UNROLL):
        k = k + cnt[u]
    cvec[...] = k
    pltpu.sync_copy(cvec, cnts_hbm.at[wid])


_sc_partials = functools.partial(
    pl.kernel,
    out_type=(
        jax.ShapeDtypeStruct((NW, L), jnp.float32),
        jax.ShapeDtypeStruct((NW, L), jnp.float32),
    ),
    mesh=plsc.VectorSubcoreMesh(core_axis_name="c", subcore_axis_name="s"),
    scratch_types=[
        pltpu.VMEM((SLAB, C), jnp.float32),
        pltpu.VMEM((SLAB, C), jnp.float32),
        pltpu.VMEM((SLAB, C), jnp.float32),
        pltpu.VMEM((SLAB, C), jnp.float32),
        pltpu.VMEM((L,), jnp.float32),
        pltpu.VMEM((L,), jnp.float32),
        pltpu.SemaphoreType.DMA,
        pltpu.SemaphoreType.DMA,
        pltpu.SemaphoreType.DMA,
        pltpu.SemaphoreType.DMA,
    ],
)(_sc_partial_kernel)


def _tc_partial_kernel(in_ref, tg_ref, sum_ref, cnt_ref):
    x = in_ref[0]
    t = tg_ref[0]
    d = t - x
    m = t >= 1.0
    s = jnp.sum(jnp.where(m, d * d, 0.0))
    k = jnp.sum(jnp.where(m, 1.0, 0.0))

    @pl.when(pl.program_id(0) == 0)
    def _():
        sum_ref[0, 0] = 0.0
        cnt_ref[0, 0] = 0.0

    sum_ref[0, 0] += s
    cnt_ref[0, 0] += k


def _tc_partials(inputs, targets):
    grid = TC_ROWS // TC_BLK
    rb = R // TC_BLK
    return pl.pallas_call(
        _tc_partial_kernel,
        grid=(grid,),
        in_specs=[
            pl.BlockSpec((1, TC_BLK, C), lambda i: (i // rb, i % rb, 0)),
            pl.BlockSpec((1, TC_BLK, C), lambda i: (i // rb, i % rb, 0)),
        ],
        out_specs=[
            pl.BlockSpec(memory_space=pltpu.SMEM),
            pl.BlockSpec(memory_space=pltpu.SMEM),
        ],
        out_shape=[
            jax.ShapeDtypeStruct((1, 1), jnp.float32),
            jax.ShapeDtypeStruct((1, 1), jnp.float32),
        ],
    )(inputs, targets)


def _combine_kernel(sums_ref, cnts_ref, tsum_ref, tcnt_ref, out_ref):
    s = jnp.sum(sums_ref[...]) + tsum_ref[0, 0]
    k = jnp.sum(cnts_ref[...]) + tcnt_ref[0, 0]
    out_ref[0, 0] = s / k


def _combine(sums, cnts, tsum, tcnt):
    return pl.pallas_call(
        _combine_kernel,
        out_shape=jax.ShapeDtypeStruct((1, 1), jnp.float32),
        in_specs=[
            pl.BlockSpec(memory_space=pltpu.VMEM),
            pl.BlockSpec(memory_space=pltpu.VMEM),
            pl.BlockSpec(memory_space=pltpu.SMEM),
            pl.BlockSpec(memory_space=pltpu.SMEM),
        ],
        out_specs=pl.BlockSpec(memory_space=pltpu.SMEM),
    )(sums, cnts, tsum, tcnt)


def kernel(inputs, targets):
    sums, cnts = _sc_partials(inputs, targets)
    tsum, tcnt = _tc_partials(inputs, targets)
    return _combine(sums, cnts, tsum, tcnt)[0, 0]


# speedup vs baseline: 1.0163x; 1.0163x over previous
import functools

import jax
import jax.numpy as jnp
from jax import lax
from jax.experimental import pallas as pl
from jax.experimental.pallas import tpu as pltpu
from jax.experimental.pallas import tpu_sc as plsc

B, R, C = 2, 8192, 2048
NC, NS, L = 2, 16, 16
NW = NC * NS
SLAB = 8
UNROLL = 8

TC_ROWS = 9728
SC_ROWS = B * R - TC_ROWS
ROWS_PER_W = SC_ROWS // NW
N_CHUNKS = ROWS_PER_W // SLAB
TC_BLK = 1024


def _sc_partial_kernel(in_hbm, tg_hbm, sums_hbm, cnts_hbm,
                       ib0, tb0, ib1, tb1, ovec, cvec,
                       si0, st0, si1, st1):
    wid = lax.axis_index("s") * NC + lax.axis_index("c")
    fr0 = TC_ROWS + wid * ROWS_PER_W

    zero = jnp.zeros((L,), jnp.float32)
    one = jnp.full((L,), 1.0, jnp.float32)
    acc0 = tuple(zero for _ in range(UNROLL))
    cnt0 = tuple(zero for _ in range(UNROLL))

    def start(c, ib, tb, semi, semt):
        fr = fr0 + c * SLAB
        b = fr // R
        r = fr % R
        pltpu.async_copy(in_hbm.at[b, pl.ds(r, SLAB), :], ib, semi)
        pltpu.async_copy(tg_hbm.at[b, pl.ds(r, SLAB), :], tb, semt)

    def wait(ib, tb, semi, semt):
        pltpu.make_async_copy(in_hbm.at[0, pl.ds(0, SLAB), :], ib, semi).wait()
        pltpu.make_async_copy(tg_hbm.at[0, pl.ds(0, SLAB), :], tb, semt).wait()

    def compute(ib, tb, acc, cnt):
        @plsc.parallel_loop(0, C, step=L, unroll=1, carry=(acc, cnt))
        def inner(j, carry):
            acc2, cnt2 = carry
            acc2 = list(acc2)
            cnt2 = list(cnt2)
            for u in range(UNROLL):
                t = tb[u, pl.ds(j, L)]
                x = ib[u, pl.ds(j, L)]
                d = t - x
                m = t >= 1.0
                acc2[u] = acc2[u] + jnp.where(m, d * d, zero)
                cnt2[u] = cnt2[u] + jnp.where(m, one, zero)
            return tuple(acc2), tuple(cnt2)

        return inner

    start(0, ib0, tb0, si0, st0)
    start(1, ib1, tb1, si1, st1)

    def pair(p, carry):
        acc, cnt = carry
        c0 = 2 * p
        wait(ib0, tb0, si0, st0)
        acc, cnt = compute(ib0, tb0, acc, cnt)

        @pl.when(c0 + 2 < N_CHUNKS)
        def _():
            start(c0 + 2, ib0, tb0, si0, st0)

        wait(ib1, tb1, si1, st1)
        acc, cnt = compute(ib1, tb1, acc, cnt)

        @pl.when(c0 + 3 < N_CHUNKS)
        def _():
            start(c0 + 3, ib1, tb1, si1, st1)

        return acc, cnt

    acc, cnt = lax.fori_loop(0, N_CHUNKS // 2, pair, (acc0, cnt0))

    s = acc[0]
    for u in range(1, UNROLL):
        s = s + acc[u]
    ovec[...] = s
    pltpu.sync_copy(ovec, sums_hbm.at[wid])

    k = cnt[0]
    for u in range(1, UNROLL):
        k = k + cnt[u]
    cvec[...] = k
    pltpu.sync_copy(cvec, cnts_hbm.at[wid])


_sc_partials = functools.partial(
    pl.kernel,
    out_type=(
        jax.ShapeDtypeStruct((NW, L), jnp.float32),
        jax.ShapeDtypeStruct((NW, L), jnp.float32),
    ),
    mesh=plsc.VectorSubcoreMesh(core_axis_name="c", subcore_axis_name="s"),
    scratch_types=[
        pltpu.VMEM((SLAB, C), jnp.float32),
        pltpu.VMEM((SLAB, C), jnp.float32),
        pltpu.VMEM((SLAB, C), jnp.float32),
        pltpu.VMEM((SLAB, C), jnp.float32),
        pltpu.VMEM((L,), jnp.float32),
        pltpu.VMEM((L,), jnp.float32),
        pltpu.SemaphoreType.DMA,
        pltpu.SemaphoreType.DMA,
        pltpu.SemaphoreType.DMA,
        pltpu.SemaphoreType.DMA,
    ],
)(_sc_partial_kernel)


def _tc_partial_kernel(in_ref, tg_ref, sum_ref, cnt_ref):
    x = in_ref[0]
    t = tg_ref[0]
    d = t - x
    m = t >= 1.0
    s = jnp.sum(jnp.where(m, d * d, 0.0))
    k = jnp.sum(jnp.where(m, 1.0, 0.0))

    @pl.when(pl.program_id(0) == 0)
    def _():
        sum_ref[0, 0] = 0.0
        cnt_ref[0, 0] = 0.0

    sum_ref[0, 0] += s
    cnt_ref[0, 0] += k


def _tc_partials(inputs, targets):
    grid = TC_ROWS // TC_BLK
    rb = R // TC_BLK
    return pl.pallas_call(
        _tc_partial_kernel,
        grid=(grid,),
        in_specs=[
            pl.BlockSpec((1, TC_BLK, C), lambda i: (i // rb, i % rb, 0)),
            pl.BlockSpec((1, TC_BLK, C), lambda i: (i // rb, i % rb, 0)),
        ],
        out_specs=[
            pl.BlockSpec(memory_space=pltpu.SMEM),
            pl.BlockSpec(memory_space=pltpu.SMEM),
        ],
        out_shape=[
            jax.ShapeDtypeStruct((1, 1), jnp.float32),
            jax.ShapeDtypeStruct((1, 1), jnp.float32),
        ],
    )(inputs, targets)


def _combine_kernel(sums_ref, cnts_ref, tsum_ref, tcnt_ref, out_ref):
    s = jnp.sum(sums_ref[...]) + tsum_ref[0, 0]
    k = jnp.sum(cnts_ref[...]) + tcnt_ref[0, 0]
    out_ref[0, 0] = s / k


def _combine(sums, cnts, tsum, tcnt):
    return pl.pallas_call(
        _combine_kernel,
        out_shape=jax.ShapeDtypeStruct((1, 1), jnp.float32),
        in_specs=[
            pl.BlockSpec(memory_space=pltpu.VMEM),
            pl.BlockSpec(memory_space=pltpu.VMEM),
            pl.BlockSpec(memory_space=pltpu.SMEM),
            pl.BlockSpec(memory_space=pltpu.SMEM),
        ],
        out_specs=pl.BlockSpec(memory_space=pltpu.SMEM),
    )(sums, cnts, tsum, tcnt)


def kernel(inputs, targets):
    sums, cnts = _sc_partials(inputs, targets)
    tsum, tcnt = _tc_partials(inputs, targets)
    return _combine(sums, cnts, tsum, tcnt)[0, 0]
